# baseline (speedup 1.0000x reference)
import jax
import jax.numpy as jnp
from jax.experimental import pallas as pl
from jax.experimental.pallas import tpu as pltpu


def _dct2_ortho_mat(N):
    n = jnp.arange(N, dtype=jnp.float32)
    k = n[:, None]
    C = jnp.cos(jnp.pi * (2.0 * n + 1.0) * k / (2.0 * N))
    scale = jnp.where(k == 0, jnp.sqrt(1.0 / N), jnp.sqrt(2.0 / N))
    return (C * scale).astype(jnp.float32)


def _pool_matrix():
    C64 = _dct2_ortho_mat(64)
    C32 = _dct2_ortho_mat(32)
    return C32.T @ C64[:32, :]


_NSLICE = 2


def _spectral_pool_kernel(x_ref, m_ref, mt_ref, o_ref):
    m = m_ref[...]
    mt = mt_ref[...]

    for s in range(_NSLICE):
        x = x_ref[0, s]
        t = jnp.dot(x.reshape(64 * 64, 64), mt,
                    preferred_element_type=jnp.float32)
        t = t.reshape(64, 64, 32).transpose(0, 2, 1)
        t = jnp.dot(t.reshape(64 * 32, 64), mt,
                    preferred_element_type=jnp.float32)
        t = t.reshape(64, 32, 32).transpose(0, 2, 1)
        t = t.reshape(64, 32 * 32)
        o_ref[s] = jnp.dot(m, t,
                           preferred_element_type=jnp.float32)


def kernel(x):
    B, C, D, H, W = x.shape
    M = _pool_matrix()
    cb = C // _NSLICE
    n = B * cb

    out = pl.pallas_call(
        _spectral_pool_kernel,
        grid=(n,),
        in_specs=[
            pl.BlockSpec((1, _NSLICE, D, H, W),
                         lambda i: (i // cb, i % cb, 0, 0, 0)),
            pl.BlockSpec((32, 64), lambda i: (0, 0)),
            pl.BlockSpec((64, 32), lambda i: (0, 0)),
        ],
        out_specs=pl.BlockSpec((_NSLICE, 32, 32 * 32), lambda i: (i, 0, 0)),
        out_shape=jax.ShapeDtypeStruct((B * C, 32, 32 * 32), jnp.float32),
        compiler_params=pltpu.CompilerParams(
            dimension_semantics=("parallel",),
        ),
    )(x, M, M.T)

    return out.reshape(B, C, 32, 32, 32)


# speedup vs baseline: 1.1919x; 1.1919x over previous
import jax
import jax.numpy as jnp
from jax.experimental import pallas as pl
from jax.experimental.pallas import tpu as pltpu


def _dct2_ortho_mat(N):
    n = jnp.arange(N, dtype=jnp.float32)
    k = n[:, None]
    C = jnp.cos(jnp.pi * (2.0 * n + 1.0) * k / (2.0 * N))
    scale = jnp.where(k == 0, jnp.sqrt(1.0 / N), jnp.sqrt(2.0 / N))
    return (C * scale).astype(jnp.float32)


def _pool_matrix():
    C64 = _dct2_ortho_mat(64)
    C32 = _dct2_ortho_mat(32)
    return C32.T @ C64[:32, :]


_NSLICE = 2


def _spectral_pool_kernel(x_ref, m_ref, mt_ref, o_ref):
    m = m_ref[...]
    mt = mt_ref[...]

    for s in range(_NSLICE):
        x = x_ref[0, s]
        t = jnp.dot(x.reshape(64 * 64, 64), mt,
                    preferred_element_type=jnp.float32)
        t = t.reshape(64, 64, 32).transpose(0, 2, 1)
        t = jnp.dot(t.reshape(64 * 32, 64), mt,
                    preferred_element_type=jnp.float32)
        t = t.reshape(64, 32, 32).transpose(0, 2, 1)
        t = t.reshape(64, 32 * 32)
        y = jnp.dot(m, t,
                    preferred_element_type=jnp.float32)
        o_ref[s] = y.reshape(32, 32, 32)


def kernel(x):
    B, C, D, H, W = x.shape
    M = _pool_matrix()
    cb = C // _NSLICE
    n = B * cb

    out = pl.pallas_call(
        _spectral_pool_kernel,
        grid=(n,),
        in_specs=[
            pl.BlockSpec((1, _NSLICE, D, H, W),
                         lambda i: (i // cb, i % cb, 0, 0, 0)),
            pl.BlockSpec((32, 64), lambda i: (0, 0)),
            pl.BlockSpec((64, 32), lambda i: (0, 0)),
        ],
        out_specs=pl.BlockSpec((_NSLICE, 32, 32, 32), lambda i: (i, 0, 0, 0)),
        out_shape=jax.ShapeDtypeStruct((B * C, 32, 32, 32), jnp.float32),
        compiler_params=pltpu.CompilerParams(
            dimension_semantics=("parallel",),
        ),
    )(x, M, M.T)

    return out.reshape(B, C, 32, 32, 32)


# speedup vs baseline: 1.6271x; 1.3651x over previous
import jax
import jax.numpy as jnp
from jax.experimental import pallas as pl
from jax.experimental.pallas import tpu as pltpu


def _dct2_ortho_mat(N):
    n = jnp.arange(N, dtype=jnp.float32)
    k = n[:, None]
    C = jnp.cos(jnp.pi * (2.0 * n + 1.0) * k / (2.0 * N))
    scale = jnp.where(k == 0, jnp.sqrt(1.0 / N), jnp.sqrt(2.0 / N))
    return (C * scale).astype(jnp.float32)


def _pool_matrix():
    C64 = _dct2_ortho_mat(64)
    C32 = _dct2_ortho_mat(32)
    return C32.T @ C64[:32, :]


_NSLICE = 4


def _spectral_pool_kernel(x_ref, m_ref, mt_ref, o_ref):
    ns = _NSLICE
    m = m_ref[...]
    mt = mt_ref[...]

    x = x_ref[0]
    t = jnp.dot(x.reshape(ns * 64 * 64, 64), mt.astype(jnp.float32),
                preferred_element_type=jnp.float32)
    t = t.astype(jnp.bfloat16)
    t = t.reshape(ns * 64, 64, 32).transpose(0, 2, 1)
    t = jnp.dot(t.reshape(ns * 64 * 32, 64), mt,
                preferred_element_type=jnp.float32)
    t = t.astype(jnp.bfloat16)
    t = t.reshape(ns * 64, 32, 32).transpose(0, 2, 1)
    t = t.reshape(ns, 64, 32 * 32)
    for s in range(ns):
        y = jnp.dot(m, t[s],
                    preferred_element_type=jnp.float32)
        o_ref[s] = y.reshape(32, 32, 32)


def kernel(x):
    B, C, D, H, W = x.shape
    M = _pool_matrix()
    cb = C // _NSLICE
    n = B * cb

    out = pl.pallas_call(
        _spectral_pool_kernel,
        grid=(n,),
        in_specs=[
            pl.BlockSpec((1, _NSLICE, D, H, W),
                         lambda i: (i // cb, i % cb, 0, 0, 0)),
            pl.BlockSpec((32, 64), lambda i: (0, 0)),
            pl.BlockSpec((64, 32), lambda i: (0, 0)),
        ],
        out_specs=pl.BlockSpec((_NSLICE, 32, 32, 32), lambda i: (i, 0, 0, 0)),
        out_shape=jax.ShapeDtypeStruct((B * C, 32, 32, 32), jnp.float32),
        compiler_params=pltpu.CompilerParams(
            dimension_semantics=("parallel",),
        ),
    )(x, M.astype(jnp.bfloat16), M.T.astype(jnp.bfloat16))

    return out.reshape(B, C, 32, 32, 32)


# speedup vs baseline: 1.6328x; 1.0035x over previous
import jax
import jax.numpy as jnp
from jax.experimental import pallas as pl
from jax.experimental.pallas import tpu as pltpu


def _dct2_ortho_mat(N):
    n = jnp.arange(N, dtype=jnp.float32)
    k = n[:, None]
    C = jnp.cos(jnp.pi * (2.0 * n + 1.0) * k / (2.0 * N))
    scale = jnp.where(k == 0, jnp.sqrt(1.0 / N), jnp.sqrt(2.0 / N))
    return (C * scale).astype(jnp.float32)


def _pool_matrix():
    C64 = _dct2_ortho_mat(64)
    C32 = _dct2_ortho_mat(32)
    return C32.T @ C64[:32, :]


_NSLICE = 4


def _spectral_pool_kernel(x_ref, m_ref, mt_ref, o_ref):
    ns = _NSLICE
    m = m_ref[...]
    mt = mt_ref[...]

    x = x_ref[0]
    t = jnp.dot(x.reshape(ns * 64 * 64, 64), mt.astype(jnp.float32),
                preferred_element_type=jnp.float32)
    t = t.astype(jnp.bfloat16)
    t = t.reshape(ns * 64, 64, 32).transpose(0, 2, 1)
    t = jnp.dot(t.reshape(ns * 64 * 32, 64), mt,
                preferred_element_type=jnp.float32)
    t = t.astype(jnp.bfloat16)
    t = t.reshape(ns * 64, 32, 32)
    for s in range(ns):
        ts = t[s * 64:(s + 1) * 64].reshape(64, 32 * 32)
        y = jnp.dot(m, ts,
                    preferred_element_type=jnp.float32)
        o_ref[s] = y.reshape(32, 32, 32).transpose(0, 2, 1)


def kernel(x):
    B, C, D, H, W = x.shape
    M = _pool_matrix()
    cb = C // _NSLICE
    n = B * cb

    out = pl.pallas_call(
        _spectral_pool_kernel,
        grid=(n,),
        in_specs=[
            pl.BlockSpec((1, _NSLICE, D, H, W),
                         lambda i: (i // cb, i % cb, 0, 0, 0)),
            pl.BlockSpec((32, 64), lambda i: (0, 0)),
            pl.BlockSpec((64, 32), lambda i: (0, 0)),
        ],
        out_specs=pl.BlockSpec((_NSLICE, 32, 32, 32), lambda i: (i, 0, 0, 0)),
        out_shape=jax.ShapeDtypeStruct((B * C, 32, 32, 32), jnp.float32),
        compiler_params=pltpu.CompilerParams(
            dimension_semantics=("parallel",),
        ),
    )(x, M.astype(jnp.bfloat16), M.T.astype(jnp.bfloat16))

    return out.reshape(B, C, 32, 32, 32)


# speedup vs baseline: 1.7907x; 1.0967x over previous
import jax
import jax.numpy as jnp
from jax.experimental import pallas as pl
from jax.experimental.pallas import tpu as pltpu


def _dct2_ortho_mat(N):
    n = jnp.arange(N, dtype=jnp.float32)
    k = n[:, None]
    C = jnp.cos(jnp.pi * (2.0 * n + 1.0) * k / (2.0 * N))
    scale = jnp.where(k == 0, jnp.sqrt(1.0 / N), jnp.sqrt(2.0 / N))
    return (C * scale).astype(jnp.float32)


def _pool_matrix():
    C64 = _dct2_ortho_mat(64)
    C32 = _dct2_ortho_mat(32)
    return C32.T @ C64[:32, :]


_NSLICE = 8


def _spectral_pool_kernel(x_ref, m_ref, mt_ref, o_ref):
    ns = _NSLICE
    m = m_ref[...]
    mt = mt_ref[...]

    x = x_ref[0]
    t = jnp.dot(x.reshape(ns * 64 * 64, 64), mt.astype(jnp.float32),
                preferred_element_type=jnp.float32)
    t = t.astype(jnp.bfloat16)
    t = t.reshape(ns * 64, 64, 32).transpose(0, 2, 1)
    t = jnp.dot(t.reshape(ns * 64 * 32, 64), mt,
                preferred_element_type=jnp.float32)
    t = t.astype(jnp.bfloat16)
    t = t.reshape(ns * 64, 32, 32)
    for s in range(ns):
        ts = t[s * 64:(s + 1) * 64].reshape(64, 32 * 32)
        y = jnp.dot(m, ts,
                    preferred_element_type=jnp.float32)
        o_ref[s] = y.reshape(32, 32, 32).transpose(0, 2, 1)


def kernel(x):
    B, C, D, H, W = x.shape
    M = _pool_matrix()
    cb = C // _NSLICE
    n = B * cb

    out = pl.pallas_call(
        _spectral_pool_kernel,
        grid=(n,),
        in_specs=[
            pl.BlockSpec((1, _NSLICE, D, H, W),
                         lambda i: (i // cb, i % cb, 0, 0, 0)),
            pl.BlockSpec((32, 64), lambda i: (0, 0)),
            pl.BlockSpec((64, 32), lambda i: (0, 0)),
        ],
        out_specs=pl.BlockSpec((_NSLICE, 32, 32, 32), lambda i: (i, 0, 0, 0)),
        out_shape=jax.ShapeDtypeStruct((B * C, 32, 32, 32), jnp.float32),
        compiler_params=pltpu.CompilerParams(
            dimension_semantics=("parallel",),
        ),
    )(x, M.astype(jnp.bfloat16), M.T.astype(jnp.bfloat16))

    return out.reshape(B, C, 32, 32, 32)


# speedup vs baseline: 1.9063x; 1.0645x over previous
import jax
import jax.numpy as jnp
from jax.experimental import pallas as pl
from jax.experimental.pallas import tpu as pltpu


def _dct2_ortho_mat(N):
    n = jnp.arange(N, dtype=jnp.float32)
    k = n[:, None]
    C = jnp.cos(jnp.pi * (2.0 * n + 1.0) * k / (2.0 * N))
    scale = jnp.where(k == 0, jnp.sqrt(1.0 / N), jnp.sqrt(2.0 / N))
    return (C * scale).astype(jnp.float32)


def _pool_matrix():
    C64 = _dct2_ortho_mat(64)
    C32 = _dct2_ortho_mat(32)
    return C32.T @ C64[:32, :]


_NSLICE = 8


def _spectral_pool_kernel(x_ref, m_ref, mt_ref, o_ref):
    ns = _NSLICE
    m = m_ref[...]
    mt = mt_ref[...]

    mtf = mt.astype(jnp.float32)
    x = x_ref[0]
    h = ns // 2
    halves = [x[:h], x[h:]]
    ts1 = []
    for xg in halves:
        t = jnp.dot(xg.reshape(h * 64 * 64, 64), mtf,
                    preferred_element_type=jnp.float32)
        ts1.append(t.astype(jnp.bfloat16))
    ts2 = []
    for t in ts1:
        t = t.reshape(h * 64, 64, 32).transpose(0, 2, 1)
        t = jnp.dot(t.reshape(h * 64 * 32, 64), mt,
                    preferred_element_type=jnp.float32)
        ts2.append(t.astype(jnp.bfloat16))
    for g, t in enumerate(ts2):
        t = t.reshape(h * 64, 32, 32)
        for s in range(h):
            ts = t[s * 64:(s + 1) * 64].reshape(64, 32 * 32)
            y = jnp.dot(m, ts,
                        preferred_element_type=jnp.float32)
            o_ref[g * h + s] = y.reshape(32, 32, 32).transpose(0, 2, 1)


def kernel(x):
    B, C, D, H, W = x.shape
    M = _pool_matrix()
    cb = C // _NSLICE
    n = B * cb

    out = pl.pallas_call(
        _spectral_pool_kernel,
        grid=(n,),
        in_specs=[
            pl.BlockSpec((1, _NSLICE, D, H, W),
                         lambda i: (i // cb, i % cb, 0, 0, 0)),
            pl.BlockSpec((32, 64), lambda i: (0, 0)),
            pl.BlockSpec((64, 32), lambda i: (0, 0)),
        ],
        out_specs=pl.BlockSpec((_NSLICE, 32, 32, 32), lambda i: (i, 0, 0, 0)),
        out_shape=jax.ShapeDtypeStruct((B * C, 32, 32, 32), jnp.float32),
        compiler_params=pltpu.CompilerParams(
            dimension_semantics=("parallel",),
        ),
    )(x, M.astype(jnp.bfloat16), M.T.astype(jnp.bfloat16))

    return out.reshape(B, C, 32, 32, 32)
